# Initial kernel scaffold; baseline (speedup 1.0000x reference)
import jax
import jax.numpy as jnp
from jax.experimental import pallas as pl


def kernel(x, w, gamma, beta):
    raise NotImplementedError("write your pallas kernel here")


# speedup vs baseline: 1.0782x; 1.0782x over previous
import jax
import jax.numpy as jnp
from jax.experimental import pallas as pl
from jax.experimental.pallas import tpu as pltpu


def _conv_im2col(xb, w_flat):
    C, H, W = xb.shape
    xbf = xb.astype(jnp.bfloat16)
    zrow = jnp.zeros((C, 1, W), jnp.bfloat16)
    rows = jnp.concatenate([zrow, xbf, zrow], axis=1)
    zcol = jnp.zeros((C, H + 2, 1), jnp.bfloat16)
    xpad = jnp.concatenate([zcol, rows, zcol], axis=2)
    patches = [
        xpad[:, kh:kh + H, kw:kw + W].reshape(C, H * W)
        for kh in range(3) for kw in range(3)
    ]
    pat = jnp.concatenate(patches, axis=0)
    return jnp.dot(w_flat, pat, preferred_element_type=jnp.float32)


def kernel(x, w, gamma, beta):
    eps = 1e-5
    N, C, H, W = x.shape
    P = H * W

    w_flat32 = jnp.transpose(w, (0, 2, 3, 1)).reshape(C, 9 * C)
    w1 = w_flat32.astype(jnp.bfloat16)

    compiler_params = pltpu.CompilerParams(
        dimension_semantics=("parallel",),
        vmem_limit_bytes=64 * 1024 * 1024,
    )

    def stats_kernel(x_ref, w_ref, part_ref):
        acc = _conv_im2col(x_ref[0], w_ref[...])
        s1 = jnp.sum(acc, axis=1, keepdims=True)
        s2 = jnp.sum(acc * acc, axis=1, keepdims=True)
        part_ref[...] = jnp.concatenate([s1, s2], axis=1)[None]

    flops1 = 2 * N * P * C * C * 9 + 3 * N * C * P
    bytes1 = 4 * (N * C * P + N * C * 2) + 2 * 9 * C * C
    part = pl.pallas_call(
        stats_kernel,
        grid=(N,),
        in_specs=[
            pl.BlockSpec((1, C, H, W), lambda n: (n, 0, 0, 0)),
            pl.BlockSpec((C, 9 * C), lambda n: (0, 0)),
        ],
        out_specs=pl.BlockSpec((1, C, 2), lambda n: (n, 0, 0)),
        out_shape=jax.ShapeDtypeStruct((N, C, 2), jnp.float32),
        compiler_params=compiler_params,
        cost_estimate=pl.CostEstimate(flops=flops1, transcendentals=0,
                                      bytes_accessed=bytes1),
    )(x, w1)

    tot = jnp.sum(part, axis=0)
    cnt = jnp.float32(N * P)
    mean = tot[:, 0] / cnt
    var = jnp.maximum(tot[:, 1] / cnt - mean * mean, 0.0)
    inv_std = jax.lax.rsqrt(var + eps)
    scale = gamma * inv_std
    shift = (beta - mean * scale).reshape(C, 1)
    w2 = (w_flat32 * scale[:, None]).astype(jnp.bfloat16)

    def apply_kernel(x_ref, w_ref, shift_ref, o_ref):
        xb = x_ref[0]
        conv = _conv_im2col(xb, w_ref[...])
        o_ref[...] = (xb.reshape(C, P) + conv + shift_ref[...])[None]

    flops2 = 2 * N * P * C * C * 9 + 2 * N * C * P
    bytes2 = 4 * (2 * N * C * P + C) + 2 * 9 * C * C
    out_flat = pl.pallas_call(
        apply_kernel,
        grid=(N,),
        in_specs=[
            pl.BlockSpec((1, C, H, W), lambda n: (n, 0, 0, 0)),
            pl.BlockSpec((C, 9 * C), lambda n: (0, 0)),
            pl.BlockSpec((C, 1), lambda n: (0, 0)),
        ],
        out_specs=pl.BlockSpec((1, C, P), lambda n: (n, 0, 0)),
        out_shape=jax.ShapeDtypeStruct((N, C, P), jnp.float32),
        compiler_params=compiler_params,
        cost_estimate=pl.CostEstimate(flops=flops2, transcendentals=0,
                                      bytes_accessed=bytes2),
    )(x, w2, shift)

    return out_flat.reshape(N, C, H, W)


# speedup vs baseline: 1.7310x; 1.6055x over previous
import jax
import jax.numpy as jnp
from jax.experimental import pallas as pl
from jax.experimental.pallas import tpu as pltpu


def _build_shift_scratch(xf, s_ref, H, W):
    C, P = xf.shape
    xb = xf.astype(jnp.bfloat16)
    col = jax.lax.broadcasted_iota(jnp.int32, (1, P), 1) % W
    zc = jnp.zeros((C, 1), jnp.bfloat16)
    xm = jnp.where(col == 0, jnp.bfloat16(0),
                   jnp.concatenate([zc, xb[:, :P - 1]], axis=1))
    xp = jnp.where(col == W - 1, jnp.bfloat16(0),
                   jnp.concatenate([xb[:, 1:], zc], axis=1))
    zpad = jnp.zeros((C, W), jnp.bfloat16)
    s_ref[0 * C:1 * C, :] = jnp.concatenate([zpad, xm, zpad], axis=1)
    s_ref[1 * C:2 * C, :] = jnp.concatenate([zpad, xb, zpad], axis=1)
    s_ref[2 * C:3 * C, :] = jnp.concatenate([zpad, xp, zpad], axis=1)


def _conv_from_scratch(s_ref, w_ref, H, W, P):
    acc = jnp.dot(w_ref[0], s_ref[:, 0:P],
                  preferred_element_type=jnp.float32)
    acc += jnp.dot(w_ref[1], s_ref[:, W:W + P],
                   preferred_element_type=jnp.float32)
    acc += jnp.dot(w_ref[2], s_ref[:, 2 * W:2 * W + P],
                   preferred_element_type=jnp.float32)
    return acc


def kernel(x, w, gamma, beta):
    eps = 1e-5
    N, C, H, W = x.shape
    P = H * W
    x_flat = x.reshape(N, C, P)

    w_kh32 = jnp.transpose(w, (2, 0, 3, 1)).reshape(3, C, 3 * C)
    w1 = w_kh32.astype(jnp.bfloat16)

    compiler_params = pltpu.CompilerParams(
        dimension_semantics=("parallel",),
        vmem_limit_bytes=64 * 1024 * 1024,
    )
    scratch = [pltpu.VMEM((3 * C, (H + 2) * W), jnp.bfloat16)]

    def stats_kernel(x_ref, w_ref, part_ref, s_ref):
        _build_shift_scratch(x_ref[0], s_ref, H, W)
        acc = _conv_from_scratch(s_ref, w_ref, H, W, P)
        s1 = jnp.sum(acc, axis=1, keepdims=True)
        s2 = jnp.sum(acc * acc, axis=1, keepdims=True)
        part_ref[...] = jnp.concatenate([s1, s2], axis=1)[None]

    flops1 = 2 * N * P * C * C * 9 + 3 * N * C * P
    bytes1 = 4 * (N * C * P + N * C * 2) + 2 * 9 * C * C
    part = pl.pallas_call(
        stats_kernel,
        grid=(N,),
        in_specs=[
            pl.BlockSpec((1, C, P), lambda n: (n, 0, 0)),
            pl.BlockSpec((3, C, 3 * C), lambda n: (0, 0, 0)),
        ],
        out_specs=pl.BlockSpec((1, C, 2), lambda n: (n, 0, 0)),
        out_shape=jax.ShapeDtypeStruct((N, C, 2), jnp.float32),
        scratch_shapes=scratch,
        compiler_params=compiler_params,
        cost_estimate=pl.CostEstimate(flops=flops1, transcendentals=0,
                                      bytes_accessed=bytes1),
    )(x_flat, w1)

    tot = jnp.sum(part, axis=0)
    cnt = jnp.float32(N * P)
    mean = tot[:, 0] / cnt
    var = jnp.maximum(tot[:, 1] / cnt - mean * mean, 0.0)
    inv_std = jax.lax.rsqrt(var + eps)
    scale = gamma * inv_std
    shift = (beta - mean * scale).reshape(C, 1)
    w2 = (w_kh32 * scale[None, :, None]).astype(jnp.bfloat16)

    def apply_kernel(x_ref, w_ref, shift_ref, o_ref, s_ref):
        xf = x_ref[0]
        _build_shift_scratch(xf, s_ref, H, W)
        conv = _conv_from_scratch(s_ref, w_ref, H, W, P)
        o_ref[...] = (xf + conv + shift_ref[...])[None]

    flops2 = 2 * N * P * C * C * 9 + 2 * N * C * P
    bytes2 = 4 * (2 * N * C * P + C) + 2 * 9 * C * C
    out_flat = pl.pallas_call(
        apply_kernel,
        grid=(N,),
        in_specs=[
            pl.BlockSpec((1, C, P), lambda n: (n, 0, 0)),
            pl.BlockSpec((3, C, 3 * C), lambda n: (0, 0, 0)),
            pl.BlockSpec((C, 1), lambda n: (0, 0)),
        ],
        out_specs=pl.BlockSpec((1, C, P), lambda n: (n, 0, 0)),
        out_shape=jax.ShapeDtypeStruct((N, C, P), jnp.float32),
        scratch_shapes=scratch,
        compiler_params=compiler_params,
        cost_estimate=pl.CostEstimate(flops=flops2, transcendentals=0,
                                      bytes_accessed=bytes2),
    )(x_flat, w2, shift)

    return out_flat.reshape(N, C, H, W)


# speedup vs baseline: 1.7333x; 1.0013x over previous
import jax
import jax.numpy as jnp
from jax.experimental import pallas as pl
from jax.experimental.pallas import tpu as pltpu


def _build_shift_scratch(xb, s_ref, H, W):
    C, P = xb.shape
    col = jax.lax.broadcasted_iota(jnp.int32, (1, P), 1) % W
    zc = jnp.zeros((C, 1), jnp.bfloat16)
    xm = jnp.where(col == 0, jnp.bfloat16(0),
                   jnp.concatenate([zc, xb[:, :P - 1]], axis=1))
    xp = jnp.where(col == W - 1, jnp.bfloat16(0),
                   jnp.concatenate([xb[:, 1:], zc], axis=1))
    zpad = jnp.zeros((C, W), jnp.bfloat16)
    s_ref[0 * C:1 * C, :] = jnp.concatenate([zpad, xm, zpad], axis=1)
    s_ref[1 * C:2 * C, :] = jnp.concatenate([zpad, xb, zpad], axis=1)
    s_ref[2 * C:3 * C, :] = jnp.concatenate([zpad, xp, zpad], axis=1)


def _conv_from_scratch(s_ref, w_ref, W, P):
    acc = jnp.dot(w_ref[0], s_ref[:, 0:P],
                  preferred_element_type=jnp.float32)
    acc += jnp.dot(w_ref[1], s_ref[:, W:W + P],
                   preferred_element_type=jnp.float32)
    acc += jnp.dot(w_ref[2], s_ref[:, 2 * W:2 * W + P],
                   preferred_element_type=jnp.float32)
    return acc


def kernel(x, w, gamma, beta):
    eps = 1e-5
    N, C, H, W = x.shape
    P = H * W
    x_flat = x.reshape(N, C, P)

    w1 = jnp.transpose(w, (2, 0, 3, 1)).reshape(3, C, 3 * C).astype(jnp.bfloat16)
    gamma_c = gamma.reshape(C, 1)
    beta_c = beta.reshape(C, 1)

    def fused_kernel(x_hbm, w_ref, g_ref, b_ref, o_hbm,
                     xres, s_ref, xin, obuf, stats, sc_ref, sh_ref,
                     in_sem, out_sem):
        ph = pl.program_id(0)
        i = pl.program_id(1)

        @pl.when(ph == 0)
        def _phase0():
            @pl.when(i == 0)
            def _():
                pltpu.make_async_copy(x_hbm.at[0], xin.at[0],
                                      in_sem.at[0]).start()

            @pl.when(i < N - 1)
            def _():
                pltpu.make_async_copy(x_hbm.at[i + 1], xin.at[(i + 1) % 2],
                                      in_sem.at[(i + 1) % 2]).start()

            pltpu.make_async_copy(x_hbm.at[i], xin.at[i % 2],
                                  in_sem.at[i % 2]).wait()
            xb = xin[i % 2].astype(jnp.bfloat16)
            xres[i] = xb
            _build_shift_scratch(xb, s_ref, H, W)
            acc = _conv_from_scratch(s_ref, w_ref, W, P)
            s1 = jnp.sum(acc, axis=1, keepdims=True)
            s2 = jnp.sum(acc * acc, axis=1, keepdims=True)
            part = jnp.concatenate([s1, s2], axis=1)

            @pl.when(i == 0)
            def _():
                stats[...] = part

            @pl.when(i > 0)
            def _():
                stats[...] = stats[...] + part

        @pl.when(ph == 1)
        def _phase1():
            @pl.when(i == 0)
            def _():
                cnt = jnp.float32(N * P)
                mean = stats[:, 0:1] / cnt
                var = jnp.maximum(stats[:, 1:2] / cnt - mean * mean, 0.0)
                inv_std = jax.lax.rsqrt(var + eps)
                sc = g_ref[...] * inv_std
                sc_ref[...] = sc
                sh_ref[...] = b_ref[...] - mean * sc

            xb = xres[i]
            _build_shift_scratch(xb, s_ref, H, W)
            acc = _conv_from_scratch(s_ref, w_ref, W, P)

            @pl.when(i >= 2)
            def _():
                pltpu.make_async_copy(obuf.at[i % 2], o_hbm.at[i - 2],
                                      out_sem.at[i % 2]).wait()

            obuf[i % 2] = (xb.astype(jnp.float32)
                           + sc_ref[...] * acc + sh_ref[...])
            pltpu.make_async_copy(obuf.at[i % 2], o_hbm.at[i],
                                  out_sem.at[i % 2]).start()

            @pl.when(i == N - 1)
            def _():
                pltpu.make_async_copy(obuf.at[(i - 1) % 2], o_hbm.at[i - 1],
                                      out_sem.at[(i - 1) % 2]).wait()
                pltpu.make_async_copy(obuf.at[i % 2], o_hbm.at[i],
                                      out_sem.at[i % 2]).wait()

    flops = 2 * (2 * N * P * C * C * 9) + 5 * N * C * P
    bytes_accessed = 4 * (2 * N * C * P + 4 * C) + 2 * 9 * C * C
    out_flat = pl.pallas_call(
        fused_kernel,
        grid=(2, N),
        in_specs=[
            pl.BlockSpec(memory_space=pl.ANY),
            pl.BlockSpec((3, C, 3 * C), lambda ph, i: (0, 0, 0)),
            pl.BlockSpec((C, 1), lambda ph, i: (0, 0)),
            pl.BlockSpec((C, 1), lambda ph, i: (0, 0)),
        ],
        out_specs=pl.BlockSpec(memory_space=pl.ANY),
        out_shape=jax.ShapeDtypeStruct((N, C, P), jnp.float32),
        scratch_shapes=[
            pltpu.VMEM((N, C, P), jnp.bfloat16),
            pltpu.VMEM((3 * C, (H + 2) * W), jnp.bfloat16),
            pltpu.VMEM((2, C, P), jnp.float32),
            pltpu.VMEM((2, C, P), jnp.float32),
            pltpu.VMEM((C, 2), jnp.float32),
            pltpu.VMEM((C, 1), jnp.float32),
            pltpu.VMEM((C, 1), jnp.float32),
            pltpu.SemaphoreType.DMA((2,)),
            pltpu.SemaphoreType.DMA((2,)),
        ],
        compiler_params=pltpu.CompilerParams(
            dimension_semantics=("arbitrary", "arbitrary"),
            vmem_limit_bytes=50 * 1024 * 1024,
        ),
        cost_estimate=pl.CostEstimate(flops=flops, transcendentals=C,
                                      bytes_accessed=bytes_accessed),
    )(x_flat, w1, gamma_c, beta_c)

    return out_flat.reshape(N, C, H, W)
